# Initial kernel scaffold; baseline (speedup 1.0000x reference)
import jax
import jax.numpy as jnp
from jax.experimental import pallas as pl


def kernel(img_feats, top_k, W1, b1, W2, b2, location_feats, gps_gallery, logit_scale):
    raise NotImplementedError("write your pallas kernel here")


# speedup vs baseline: 7.7309x; 7.7309x over previous
import jax
import jax.numpy as jnp
from jax import lax
from jax.experimental import pallas as pl
from jax.experimental.pallas import tpu as pltpu
from jax.experimental.pallas import tpu_sc as plsc

K = 100000
D_OUT = 512
BK = 2048
NBLK = 49
K_PAD = NBLK * BK
NW = 16
C = K_PAD // NW
VB = C // 16
TOPK = 10
NEG = -1e30
IBIG = 2147483647


def _tc_logits_body(x_ref, w1_ref, b1_ref, w2_ref, b2_ref, s_ref, loc_ref,
                    out_ref, q_ref):
    i = pl.program_id(0)

    @pl.when(i == 0)
    def _():
        h = jnp.maximum(
            jnp.dot(x_ref[...], w1_ref[...],
                    preferred_element_type=jnp.float32) + b1_ref[...], 0.0)
        f = jnp.dot(h, w2_ref[...],
                    preferred_element_type=jnp.float32) + b2_ref[...]
        nrm = jnp.maximum(jnp.sqrt(jnp.sum(f * f)), 1e-12)
        q_ref[...] = f * (jnp.exp(s_ref[0, 0]) / nrm)

    out_ref[...] = lax.dot_general(
        q_ref[...], loc_ref[...], (((1,), (1,)), ((), ())),
        preferred_element_type=jnp.float32)


def _tc_logits(x, w1, b1, w2, b2, scale, loc):
    return pl.pallas_call(
        _tc_logits_body,
        grid=(NBLK,),
        in_specs=[
            pl.BlockSpec((1, 768), lambda i: (0, 0)),
            pl.BlockSpec((768, 768), lambda i: (0, 0)),
            pl.BlockSpec((1, 768), lambda i: (0, 0)),
            pl.BlockSpec((768, D_OUT), lambda i: (0, 0)),
            pl.BlockSpec((1, D_OUT), lambda i: (0, 0)),
            pl.BlockSpec((1, 1), lambda i: (0, 0)),
            pl.BlockSpec((BK, D_OUT), lambda i: (i, 0)),
        ],
        out_specs=pl.BlockSpec((1, BK), lambda i: (0, i)),
        out_shape=jax.ShapeDtypeStruct((1, K_PAD), jnp.float32),
        scratch_shapes=[pltpu.VMEM((1, D_OUT), jnp.float32)],
        compiler_params=pltpu.CompilerParams(
            dimension_semantics=("arbitrary",)),
    )(x, w1, b1, w2, b2, scale, loc)


def _sc_body(logits_hbm, gps_hbm, out_gps_hbm, out_prob_hbm,
             buf, vals_buf, idx_buf, ms_buf,
             mvals, midx, mms, prob_buf, rows_v,
             sh_vals, sh_idx, sh_ms, sem):
    wid = lax.axis_index("s")
    base = wid * C
    iot = lax.iota(jnp.int32, 16)

    pltpu.sync_copy(logits_hbm.at[pl.ds(base, C)], buf)

    def p_mask(j, m_vec):
        v = buf[pl.ds(j * 16, 16)]
        gidx = base + j * 16 + iot
        v = jnp.where(gidx < K, v, NEG)
        buf[pl.ds(j * 16, 16)] = v
        return jnp.maximum(m_vec, v)

    m_vec = lax.fori_loop(0, VB, p_mask, jnp.full((16,), NEG, jnp.float32))
    m_w = jnp.max(m_vec)

    def p_sum(j, s_vec):
        v = buf[pl.ds(j * 16, 16)]
        return s_vec + jnp.exp(v - m_w)

    s_vec = lax.fori_loop(0, VB, p_sum, jnp.zeros((16,), jnp.float32))

    vals_vec = jnp.full((16,), NEG, jnp.float32)
    idx_vec = jnp.zeros((16,), jnp.int32)
    for i in range(TOPK):
        def p_top(j, carry):
            mx, mi = carry
            v = buf[pl.ds(j * 16, 16)]
            gidx = base + j * 16 + iot
            c = v > mx
            return jnp.where(c, v, mx), jnp.where(c, gidx, mi)

        mx, mi = lax.fori_loop(
            0, VB, p_top,
            (jnp.full((16,), NEG, jnp.float32), jnp.zeros((16,), jnp.int32)))
        gm = jnp.max(mx)
        gi = jnp.min(jnp.where(mx == gm, mi, IBIG))
        vals_vec = jnp.where(iot == i, gm, vals_vec)
        idx_vec = jnp.where(iot == i, gi, idx_vec)
        lo = gi - base
        j0 = lo & ~15
        vv = buf[pl.ds(j0, 16)]
        buf[pl.ds(j0, 16)] = jnp.where(iot == (lo & 15), NEG, vv)

    vals_buf[...] = vals_vec
    idx_buf[...] = idx_vec
    ms_buf[0, :] = jnp.broadcast_to(m_w, (16,))
    ms_buf[1, :] = s_vec

    pltpu.sync_copy(vals_buf, sh_vals.at[pl.ds(wid * 16, 16)])
    pltpu.sync_copy(idx_buf, sh_idx.at[pl.ds(wid * 16, 16)])
    pltpu.sync_copy(ms_buf, sh_ms.at[wid])
    plsc.subcore_barrier()

    @pl.when(wid == 0)
    def _merge():
        pltpu.sync_copy(sh_vals, mvals)
        pltpu.sync_copy(sh_idx, midx)
        pltpu.sync_copy(sh_ms, mms)

        m_all = jnp.full((16,), NEG, jnp.float32)
        for w in range(NW):
            m_all = jnp.maximum(m_all, mms[w, 0, :])
        s_all = jnp.zeros((16,), jnp.float32)
        for w in range(NW):
            s_all = s_all + mms[w, 1, :] * jnp.exp(mms[w, 0, :] - m_all)
        s_tot = jnp.sum(s_all)

        tvals = jnp.full((16,), NEG, jnp.float32)
        tidx = jnp.zeros((16,), jnp.int32)
        for i in range(TOPK):
            mx = jnp.full((16,), NEG, jnp.float32)
            gx = jnp.zeros((16,), jnp.int32)
            cp = jnp.zeros((16,), jnp.int32)
            for w in range(NW):
                v = mvals[pl.ds(w * 16, 16)]
                c = v > mx
                mx = jnp.where(c, v, mx)
                gx = jnp.where(c, midx[pl.ds(w * 16, 16)], gx)
                cp = jnp.where(c, w * 16 + iot, cp)
            gm = jnp.max(mx)
            gi = jnp.min(jnp.where(mx == gm, gx, IBIG))
            cpw = jnp.min(jnp.where((mx == gm) & (gx == gi), cp, IBIG))
            tvals = jnp.where(iot == i, gm, tvals)
            tidx = jnp.where(iot == i, gi, tidx)
            j0 = cpw & ~15
            vv = mvals[pl.ds(j0, 16)]
            mvals[pl.ds(j0, 16)] = jnp.where(iot == (cpw & 15), NEG, vv)

        prob_buf[...] = jnp.exp(tvals - m_all) / s_tot
        pltpu.sync_copy(prob_buf, out_prob_hbm)

        idx_buf[...] = tidx
        pltpu.async_copy(gps_hbm.at[idx_buf], rows_v, sem).wait()
        pltpu.sync_copy(rows_v, out_gps_hbm)


def _sc_topk(logits, gps_pad):
    mesh = plsc.VectorSubcoreMesh(
        core_axis_name="c", subcore_axis_name="s", num_cores=1)
    f32 = jnp.float32
    return pl.kernel(
        _sc_body,
        out_type=[
            jax.ShapeDtypeStruct((16, 16), f32),
            jax.ShapeDtypeStruct((16,), f32),
        ],
        mesh=mesh,
        scratch_types=[
            pltpu.VMEM((C,), f32),
            pltpu.VMEM((16,), f32),
            pltpu.VMEM((16,), jnp.int32),
            pltpu.VMEM((2, 16), f32),
            pltpu.VMEM((NW * 16,), f32),
            pltpu.VMEM((NW * 16,), jnp.int32),
            pltpu.VMEM((NW, 2, 16), f32),
            pltpu.VMEM((16,), f32),
            pltpu.VMEM((16, 16), f32),
            pltpu.VMEM_SHARED((NW * 16,), f32),
            pltpu.VMEM_SHARED((NW * 16,), jnp.int32),
            pltpu.VMEM_SHARED((NW, 2, 16), f32),
            pltpu.SemaphoreType.DMA,
        ],
        compiler_params=pltpu.CompilerParams(
            needs_layout_passes=False, use_tc_tiling_on_sc=False),
    )(logits, gps_pad)


def kernel(img_feats, top_k, W1, b1, W2, b2, location_feats, gps_gallery,
           logit_scale):
    x0 = img_feats[0:1]
    b1r = b1.reshape(1, -1)
    b2r = b2.reshape(1, -1)
    scale = logit_scale.reshape(1, 1)
    logits = _tc_logits(x0, W1, b1r, W2, b2r, scale, location_feats)
    gps_pad = jnp.pad(gps_gallery, ((0, 0), (0, 14)))
    out_gps, out_prob = _sc_topk(logits.reshape(K_PAD), gps_pad)
    return out_gps[:TOPK, :2], out_prob[:TOPK]


# speedup vs baseline: 8.1947x; 1.0600x over previous
import jax
import jax.numpy as jnp
from jax import lax
from jax.experimental import pallas as pl
from jax.experimental.pallas import tpu as pltpu
from jax.experimental.pallas import tpu_sc as plsc

K = 100000
D_OUT = 512
BK = 4096
NBLK = 25
K_PAD = NBLK * BK
NW = 16
C = K_PAD // NW
VB = C // 16
TOPK = 10
NEG = -1e30
IBIG = 2147483647


def _tc_logits_body(x_ref, w1_ref, b1_ref, w2_ref, b2_ref, s_ref, loc_ref,
                    out_ref, q_ref):
    i = pl.program_id(0)

    @pl.when(i == 0)
    def _():
        h = jnp.maximum(
            jnp.dot(x_ref[...], w1_ref[...],
                    preferred_element_type=jnp.float32) + b1_ref[...], 0.0)
        f = jnp.dot(h, w2_ref[...],
                    preferred_element_type=jnp.float32) + b2_ref[...]
        nrm = jnp.maximum(jnp.sqrt(jnp.sum(f * f)), 1e-12)
        q_ref[...] = f * (jnp.exp(s_ref[0, 0]) / nrm)

    out_ref[...] = lax.dot_general(
        q_ref[...], loc_ref[...], (((1,), (1,)), ((), ())),
        preferred_element_type=jnp.float32)


def _tc_logits(x, w1, b1, w2, b2, scale, loc):
    return pl.pallas_call(
        _tc_logits_body,
        grid=(NBLK,),
        in_specs=[
            pl.BlockSpec((1, 768), lambda i: (0, 0)),
            pl.BlockSpec((768, 768), lambda i: (0, 0)),
            pl.BlockSpec((1, 768), lambda i: (0, 0)),
            pl.BlockSpec((768, D_OUT), lambda i: (0, 0)),
            pl.BlockSpec((1, D_OUT), lambda i: (0, 0)),
            pl.BlockSpec((1, 1), lambda i: (0, 0)),
            pl.BlockSpec((BK, D_OUT), lambda i: (i, 0)),
        ],
        out_specs=pl.BlockSpec((1, BK), lambda i: (0, i)),
        out_shape=jax.ShapeDtypeStruct((1, K_PAD), jnp.float32),
        scratch_shapes=[pltpu.VMEM((1, D_OUT), jnp.float32)],
        compiler_params=pltpu.CompilerParams(
            dimension_semantics=("arbitrary",)),
    )(x, w1, b1, w2, b2, scale, loc)


def _sc_body(logits_hbm, gps_hbm, out_gps_hbm, out_prob_hbm,
             buf, vals_buf, idx_buf, ms_buf,
             mvals, midx, mms, prob_buf, rows_v,
             sh_vals, sh_idx, sh_ms, sem):
    wid = lax.axis_index("s")
    base = wid * C
    iot = lax.iota(jnp.int32, 16)

    pltpu.sync_copy(logits_hbm.at[pl.ds(base, C)], buf)

    def p_mask(j, m_vec):
        v = buf[pl.ds(j * 16, 16)]
        gidx = base + j * 16 + iot
        v = jnp.where(gidx < K, v, NEG)
        buf[pl.ds(j * 16, 16)] = v
        return jnp.maximum(m_vec, v)

    m_vec = lax.fori_loop(0, VB, p_mask, jnp.full((16,), NEG, jnp.float32))
    m_w = jnp.max(m_vec)

    def p_sum(j, s_vec):
        v = buf[pl.ds(j * 16, 16)]
        return s_vec + jnp.exp(v - m_w)

    s_vec = lax.fori_loop(0, VB, p_sum, jnp.zeros((16,), jnp.float32))

    vals_vec = jnp.full((16,), NEG, jnp.float32)
    idx_vec = jnp.zeros((16,), jnp.int32)
    for i in range(TOPK):
        def p_top(j, carry):
            mx, mi = carry
            v = buf[pl.ds(j * 16, 16)]
            gidx = base + j * 16 + iot
            c = v > mx
            return jnp.where(c, v, mx), jnp.where(c, gidx, mi)

        mx, mi = lax.fori_loop(
            0, VB, p_top,
            (jnp.full((16,), NEG, jnp.float32), jnp.zeros((16,), jnp.int32)))
        gm = jnp.max(mx)
        gi = jnp.min(jnp.where(mx == gm, mi, IBIG))
        vals_vec = jnp.where(iot == i, gm, vals_vec)
        idx_vec = jnp.where(iot == i, gi, idx_vec)
        lo = gi - base
        j0 = lo & ~15
        vv = buf[pl.ds(j0, 16)]
        buf[pl.ds(j0, 16)] = jnp.where(iot == (lo & 15), NEG, vv)

    vals_buf[...] = vals_vec
    idx_buf[...] = idx_vec
    ms_buf[0, :] = jnp.broadcast_to(m_w, (16,))
    ms_buf[1, :] = s_vec

    pltpu.sync_copy(vals_buf, sh_vals.at[pl.ds(wid * 16, 16)])
    pltpu.sync_copy(idx_buf, sh_idx.at[pl.ds(wid * 16, 16)])
    pltpu.sync_copy(ms_buf, sh_ms.at[wid])
    plsc.subcore_barrier()

    @pl.when(wid == 0)
    def _merge():
        pltpu.sync_copy(sh_vals, mvals)
        pltpu.sync_copy(sh_idx, midx)
        pltpu.sync_copy(sh_ms, mms)

        m_all = jnp.full((16,), NEG, jnp.float32)
        for w in range(NW):
            m_all = jnp.maximum(m_all, mms[w, 0, :])
        s_all = jnp.zeros((16,), jnp.float32)
        for w in range(NW):
            s_all = s_all + mms[w, 1, :] * jnp.exp(mms[w, 0, :] - m_all)
        s_tot = jnp.sum(s_all)

        tvals = jnp.full((16,), NEG, jnp.float32)
        tidx = jnp.zeros((16,), jnp.int32)
        for i in range(TOPK):
            mx = jnp.full((16,), NEG, jnp.float32)
            gx = jnp.zeros((16,), jnp.int32)
            cp = jnp.zeros((16,), jnp.int32)
            for w in range(NW):
                v = mvals[pl.ds(w * 16, 16)]
                c = v > mx
                mx = jnp.where(c, v, mx)
                gx = jnp.where(c, midx[pl.ds(w * 16, 16)], gx)
                cp = jnp.where(c, w * 16 + iot, cp)
            gm = jnp.max(mx)
            gi = jnp.min(jnp.where(mx == gm, gx, IBIG))
            cpw = jnp.min(jnp.where((mx == gm) & (gx == gi), cp, IBIG))
            tvals = jnp.where(iot == i, gm, tvals)
            tidx = jnp.where(iot == i, gi, tidx)
            j0 = cpw & ~15
            vv = mvals[pl.ds(j0, 16)]
            mvals[pl.ds(j0, 16)] = jnp.where(iot == (cpw & 15), NEG, vv)

        prob_buf[...] = jnp.exp(tvals - m_all) / s_tot
        pltpu.sync_copy(prob_buf, out_prob_hbm)

        idx_buf[...] = tidx
        pltpu.async_copy(gps_hbm.at[idx_buf], rows_v, sem).wait()
        pltpu.sync_copy(rows_v, out_gps_hbm)


def _sc_topk(logits, gps_pad):
    mesh = plsc.VectorSubcoreMesh(
        core_axis_name="c", subcore_axis_name="s", num_cores=1)
    f32 = jnp.float32
    return pl.kernel(
        _sc_body,
        out_type=[
            jax.ShapeDtypeStruct((16, 16), f32),
            jax.ShapeDtypeStruct((16,), f32),
        ],
        mesh=mesh,
        scratch_types=[
            pltpu.VMEM((C,), f32),
            pltpu.VMEM((16,), f32),
            pltpu.VMEM((16,), jnp.int32),
            pltpu.VMEM((2, 16), f32),
            pltpu.VMEM((NW * 16,), f32),
            pltpu.VMEM((NW * 16,), jnp.int32),
            pltpu.VMEM((NW, 2, 16), f32),
            pltpu.VMEM((16,), f32),
            pltpu.VMEM((16, 16), f32),
            pltpu.VMEM_SHARED((NW * 16,), f32),
            pltpu.VMEM_SHARED((NW * 16,), jnp.int32),
            pltpu.VMEM_SHARED((NW, 2, 16), f32),
            pltpu.SemaphoreType.DMA,
        ],
        compiler_params=pltpu.CompilerParams(
            needs_layout_passes=False, use_tc_tiling_on_sc=False),
    )(logits, gps_pad)


def kernel(img_feats, top_k, W1, b1, W2, b2, location_feats, gps_gallery,
           logit_scale):
    x0 = img_feats[0:1]
    b1r = b1.reshape(1, -1)
    b2r = b2.reshape(1, -1)
    scale = logit_scale.reshape(1, 1)
    logits = _tc_logits(x0, W1, b1r, W2, b2r, scale, location_feats)
    gps_pad = jnp.pad(gps_gallery, ((0, 0), (0, 14)))
    out_gps, out_prob = _sc_topk(logits.reshape(K_PAD), gps_pad)
    return out_gps[:TOPK, :2], out_prob[:TOPK]
